# Initial kernel scaffold; baseline (speedup 1.0000x reference)
import jax
import jax.numpy as jnp
from jax.experimental import pallas as pl


def kernel(x, edge_index, batch, W1, b1, W2, b2, W3, b3, L1W, L1b, L2W, L2b, L3W, L3b, CW, Cb):
    raise NotImplementedError("write your pallas kernel here")


# speedup vs baseline: 14.5626x; 14.5626x over previous
import functools

import jax
import jax.numpy as jnp
from jax import lax
from jax.experimental import pallas as pl
from jax.experimental.pallas import tpu as pltpu
from jax.experimental.pallas import tpu_sc as plsc

N = 10000
E = 320000
F_IN = 128

NC = 2
NS = 16
NW = NC * NS

K = 128
BLOCKS = 80
EPT = BLOCKS * K
EP = NW * EPT
R = 10112
ZCH = R // NS
DP = 32
HP = 16

_f32 = jnp.float32


def _mesh():
    return plsc.VectorSubcoreMesh(
        core_axis_name="c", subcore_axis_name="s", num_cores=NC, num_subcores=NS
    )


def _sc_degree(dst2d):
    def body(dst_hbm, out_hbm, didx, ones_v, zbuf, acc, sem):
        c = lax.axis_index("c")
        s = lax.axis_index("s")
        wid = s * NC + c

        def zrow(i, carry):
            zbuf[pl.ds(i * 16, 16)] = jnp.zeros((16,), _f32)
            return carry

        lax.fori_loop(0, ZCH // 16, zrow, 0)
        zbuf[pl.ds(ZCH - 16, 16)] = jnp.zeros((16,), _f32)
        pltpu.sync_copy(zbuf, acc.at[pl.ds(s * ZCH, ZCH)])
        for i in range(K // 16):
            ones_v[pl.ds(i * 16, 16)] = jnp.ones((16,), _f32)
        plsc.subcore_barrier()

        base = wid * BLOCKS
        pltpu.sync_copy(dst_hbm.at[pl.ds(base, BLOCKS)], didx)
        for j in range(BLOCKS):
            pltpu.sync_copy(ones_v, acc.at[didx.at[j]], add=True)
        plsc.subcore_barrier()
        pltpu.sync_copy(acc.at[pl.ds(s * ZCH, ZCH)], zbuf)
        pltpu.sync_copy(zbuf, out_hbm.at[pl.ds(c * R + s * ZCH, ZCH)])

    fn = pl.kernel(
        body,
        out_type=jax.ShapeDtypeStruct((NC * R,), _f32),
        mesh=_mesh(),
        compiler_params=pltpu.CompilerParams(use_tc_tiling_on_sc=False),
        scratch_types=[
            pltpu.VMEM((BLOCKS, K), jnp.int32),
            pltpu.VMEM((K,), _f32),
            pltpu.VMEM((ZCH,), _f32),
            pltpu.VMEM_SHARED((R,), _f32),
            pltpu.SemaphoreType.DMA,
        ],
    )
    return fn(dst2d)


def _sc_aggregate(g, src2d, dst2d):
    def body(g_hbm, src_hbm, dst_hbm, out_hbm, sidx, didx, buf0, buf1, zbuf,
             acc, sem0, sem1):
        c = lax.axis_index("c")
        s = lax.axis_index("s")
        wid = s * NC + c

        def zrow(i, carry):
            zbuf[i, pl.ds(0, 16)] = jnp.zeros((16,), _f32)
            zbuf[i, pl.ds(16, 16)] = jnp.zeros((16,), _f32)
            return carry

        lax.fori_loop(0, ZCH, zrow, 0)
        pltpu.sync_copy(zbuf, acc.at[pl.ds(s * ZCH, ZCH)])
        plsc.subcore_barrier()

        base = wid * BLOCKS
        pltpu.sync_copy(src_hbm.at[pl.ds(base, BLOCKS)], sidx)
        pltpu.sync_copy(dst_hbm.at[pl.ds(base, BLOCKS)], didx)

        bufs = (buf0, buf1)
        sems = (sem0, sem1)
        cps = [None, None]
        cps[0] = pltpu.async_copy(g_hbm.at[sidx.at[0]], buf0, sem0)
        for j in range(BLOCKS):
            nxt = (j + 1) % 2
            if j + 1 < BLOCKS:
                cps[nxt] = pltpu.async_copy(
                    g_hbm.at[sidx.at[j + 1]], bufs[nxt], sems[nxt]
                )
            cps[j % 2].wait()
            pltpu.sync_copy(bufs[j % 2], acc.at[didx.at[j]], add=True)
        plsc.subcore_barrier()
        pltpu.sync_copy(acc.at[pl.ds(s * ZCH, ZCH)], zbuf)
        pltpu.sync_copy(zbuf, out_hbm.at[c, pl.ds(s * ZCH, ZCH)])

    fn = pl.kernel(
        body,
        out_type=jax.ShapeDtypeStruct((NC, R, DP), _f32),
        mesh=_mesh(),
        compiler_params=pltpu.CompilerParams(use_tc_tiling_on_sc=False),
        scratch_types=[
            pltpu.VMEM((BLOCKS, K), jnp.int32),
            pltpu.VMEM((BLOCKS, K), jnp.int32),
            pltpu.VMEM((K, DP), _f32),
            pltpu.VMEM((K, DP), _f32),
            pltpu.VMEM((ZCH, DP), _f32),
            pltpu.VMEM_SHARED((R, DP), _f32),
            pltpu.SemaphoreType.DMA,
            pltpu.SemaphoreType.DMA,
        ],
    )
    return fn(g, src2d, dst2d)


def _sc_edge_gather(h, src2d, dst2d):
    def body(h_hbm, src_hbm, dst_hbm, es_hbm, ed_hbm, sidx, didx,
             sb0, sb1, db0, db1, sem0, sem1, sem2, sem3):
        c = lax.axis_index("c")
        s = lax.axis_index("s")
        wid = s * NC + c
        base = wid * BLOCKS
        pltpu.sync_copy(src_hbm.at[pl.ds(base, BLOCKS)], sidx)
        pltpu.sync_copy(dst_hbm.at[pl.ds(base, BLOCKS)], didx)

        sbufs = (sb0, sb1)
        dbufs = (db0, db1)
        ssems = (sem0, sem1)
        dsems = (sem2, sem3)
        scp = [None, None]
        dcp = [None, None]
        scp[0] = pltpu.async_copy(h_hbm.at[sidx.at[0]], sb0, sem0)
        dcp[0] = pltpu.async_copy(h_hbm.at[didx.at[0]], db0, sem2)
        for j in range(BLOCKS):
            nxt = (j + 1) % 2
            if j + 1 < BLOCKS:
                scp[nxt] = pltpu.async_copy(
                    h_hbm.at[sidx.at[j + 1]], sbufs[nxt], ssems[nxt]
                )
                dcp[nxt] = pltpu.async_copy(
                    h_hbm.at[didx.at[j + 1]], dbufs[nxt], dsems[nxt]
                )
            scp[j % 2].wait()
            dcp[j % 2].wait()
            row = (base + j) * K
            pltpu.sync_copy(sbufs[j % 2], es_hbm.at[pl.ds(row, K)])
            pltpu.sync_copy(dbufs[j % 2], ed_hbm.at[pl.ds(row, K)])

    fn = pl.kernel(
        body,
        out_type=(
            jax.ShapeDtypeStruct((EP, HP), _f32),
            jax.ShapeDtypeStruct((EP, HP), _f32),
        ),
        mesh=_mesh(),
        compiler_params=pltpu.CompilerParams(use_tc_tiling_on_sc=False),
        scratch_types=[
            pltpu.VMEM((BLOCKS, K), jnp.int32),
            pltpu.VMEM((BLOCKS, K), jnp.int32),
            pltpu.VMEM((K, HP), _f32),
            pltpu.VMEM((K, HP), _f32),
            pltpu.VMEM((K, HP), _f32),
            pltpu.VMEM((K, HP), _f32),
            pltpu.SemaphoreType.DMA,
            pltpu.SemaphoreType.DMA,
            pltpu.SemaphoreType.DMA,
            pltpu.SemaphoreType.DMA,
        ],
    )
    return fn(h, src2d, dst2d)


def _tc_call(body, out_shapes, *args):
    return pl.pallas_call(
        body,
        out_shape=out_shapes,
    )(*args)


def _tc_pre(deg_p, x, w1p):
    def body(deg_ref, x_ref, w_ref, dinv_ref, t0_ref, g0_ref):
        deg = deg_ref[0, :] + deg_ref[1, :] + 1.0
        dinv = lax.rsqrt(deg).reshape(R, 1)
        dinv_ref[...] = dinv
        t0 = jnp.dot(x_ref[...], w_ref[...], preferred_element_type=_f32)
        t0_ref[...] = t0
        g0 = dinv[:N] * t0
        g0_ref[...] = jnp.concatenate([g0, jnp.zeros((R - N, DP), _f32)], axis=0)

    return _tc_call(
        body,
        (
            jax.ShapeDtypeStruct((R, 1), _f32),
            jax.ShapeDtypeStruct((N, DP), _f32),
            jax.ShapeDtypeStruct((R, DP), _f32),
        ),
        deg_p, x, w1p,
    )


def _tc_layer1(agg_p, t0, dinv, b1p):
    def body(agg_ref, t0_ref, dinv_ref, b_ref, h1_ref, g1_ref):
        dv = dinv_ref[:N]
        sagg = agg_ref[0, :N] + agg_ref[1, :N]
        h1 = jnp.tanh(dv * sagg + (dv * dv) * t0_ref[...] + b_ref[...])
        h1_ref[...] = h1
        g1_ref[...] = jnp.concatenate(
            [dv * h1, jnp.zeros((R - N, DP), _f32)], axis=0
        )

    return _tc_call(
        body,
        (
            jax.ShapeDtypeStruct((N, DP), _f32),
            jax.ShapeDtypeStruct((R, DP), _f32),
        ),
        agg_p, t0, dinv, b1p,
    )


def _tc_layer2(agg_p, h1, dinv, w2p, b2p):
    def body(agg_ref, h1_ref, dinv_ref, w_ref, b_ref, h2_ref, g2_ref):
        dv = dinv_ref[:N]
        sagg = agg_ref[0, :N] + agg_ref[1, :N]
        u = dv * sagg + (dv * dv) * h1_ref[...]
        h2 = jnp.tanh(
            jnp.dot(u, w_ref[...], preferred_element_type=_f32) + b_ref[...]
        )
        h2_ref[...] = h2
        g2_ref[...] = jnp.concatenate(
            [dv * h2, jnp.zeros((R - N, DP), _f32)], axis=0
        )

    return _tc_call(
        body,
        (
            jax.ShapeDtypeStruct((N, DP), _f32),
            jax.ShapeDtypeStruct((R, DP), _f32),
        ),
        agg_p, h1, dinv, w2p, b2p,
    )


def _tc_dense(agg_p, h2, dinv, w3p, b3, l1w, l1b, l2w, l2b, l3w, l3b):
    def body(agg_ref, h2_ref, dinv_ref, w3_ref, b3_ref, l1w_ref, l1b_ref,
             l2w_ref, l2b_ref, l3w_ref, l3b_ref, h6_ref):
        dv = dinv_ref[:N]
        sagg = agg_ref[0, :N] + agg_ref[1, :N]
        u = dv * sagg + (dv * dv) * h2_ref[...]
        h3 = jnp.tanh(
            jnp.dot(u, w3_ref[...], preferred_element_type=_f32) + b3_ref[...]
        )
        h4 = jnp.tanh(
            jnp.dot(h3, l1w_ref[...], preferred_element_type=_f32) + l1b_ref[...]
        )
        h5 = jnp.tanh(
            jnp.dot(h4, l2w_ref[...], preferred_element_type=_f32) + l2b_ref[...]
        )
        h6 = jnp.tanh(
            jnp.dot(h5, l3w_ref[...], preferred_element_type=_f32) + l3b_ref[...]
        )
        h6p = jnp.concatenate([h6, jnp.zeros((N, HP - 12), _f32)], axis=1)
        h6_ref[...] = jnp.concatenate(
            [h6p, jnp.zeros((R - N, HP), _f32)], axis=0
        )

    return _tc_call(
        body,
        jax.ShapeDtypeStruct((R, HP), _f32),
        agg_p, h2, dinv, w3p, b3, l1w, l1b, l2w, l2b, l3w, l3b,
    )


_FBLK = 3200


def _tc_edge_out(esrc, edst, cw, cb):
    def body(es_ref, ed_ref, cw_ref, cb_ref, e_ref, out_ref):
        eb = jnp.concatenate([es_ref[:, :12], ed_ref[:, :12]], axis=1)
        e_ref[...] = eb
        out_ref[...] = (
            jnp.dot(eb, cw_ref[...], preferred_element_type=_f32) + cb_ref[...]
        )

    grid = (E // _FBLK,)
    return pl.pallas_call(
        body,
        grid=grid,
        in_specs=[
            pl.BlockSpec((_FBLK, HP), lambda i: (i, 0)),
            pl.BlockSpec((_FBLK, HP), lambda i: (i, 0)),
            pl.BlockSpec((24, 10), lambda i: (0, 0)),
            pl.BlockSpec((1, 10), lambda i: (0, 0)),
        ],
        out_specs=[
            pl.BlockSpec((_FBLK, 24), lambda i: (i, 0)),
            pl.BlockSpec((_FBLK, 10), lambda i: (i, 0)),
        ],
        out_shape=[
            jax.ShapeDtypeStruct((E, 24), _f32),
            jax.ShapeDtypeStruct((E, 10), _f32),
        ],
    )(esrc, edst, cw, cb)


def kernel(x, edge_index, batch, W1, b1, W2, b2, W3, b3,
           L1W, L1b, L2W, L2b, L3W, L3b, CW, Cb):
    del batch
    pad = (N + (jnp.arange(EP - E, dtype=jnp.int32) % 16)).astype(jnp.int32)
    src2d = jnp.concatenate([edge_index[0], pad]).reshape(EP // K, K)
    dst2d = jnp.concatenate([edge_index[1], pad]).reshape(EP // K, K)

    w1p = jnp.pad(W1, ((0, 0), (0, DP - 18)))
    b1p = jnp.pad(b1, (0, DP - 18)).reshape(1, DP)
    w2p = jnp.pad(W2, ((0, DP - 18), (0, DP - 24)))
    b2p = jnp.pad(b2, (0, DP - 24)).reshape(1, DP)
    w3p = jnp.pad(W3, ((0, DP - 24), (0, 0)))
    b3r = b3.reshape(1, 2 * F_IN)
    l1b = L1b.reshape(1, 24)
    l2b = L2b.reshape(1, 18)
    l3b = L3b.reshape(1, 12)
    cbr = Cb.reshape(1, 10)

    deg_p = _sc_degree(dst2d).reshape(NC, R)
    dinv, t0, g0 = _tc_pre(deg_p, x, w1p)
    agg0 = _sc_aggregate(g0, src2d, dst2d)
    h1, g1 = _tc_layer1(agg0, t0, dinv, b1p)
    agg1 = _sc_aggregate(g1, src2d, dst2d)
    h2, g2 = _tc_layer2(agg1, h1, dinv, w2p, b2p)
    agg2 = _sc_aggregate(g2, src2d, dst2d)
    h6 = _tc_dense(agg2, h2, dinv, w3p, b3r, L1W, l1b, L2W, l2b, L3W, l3b)
    esrc, edst = _sc_edge_gather(h6, src2d, dst2d)
    e, out = _tc_edge_out(esrc, edst, CW, cbr)
    return (out, e)
